# Initial kernel scaffold; baseline (speedup 1.0000x reference)
import jax
import jax.numpy as jnp
from jax.experimental import pallas as pl


def kernel(node_sca, node_vec, node_pos, edge_feature, edge_index, gds_dist, params):
    raise NotImplementedError("write your pallas kernel here")


# speedup vs baseline: 6.3009x; 6.3009x over previous
import functools
import math

import jax
import jax.numpy as jnp
import numpy as np
from jax.experimental import pallas as pl
from jax.experimental.pallas import tpu as pltpu

_CUTOFF = 10.0
_NUM_GAUSS = 14
_N_BLOCK = 2000
_E_BLOCK = 4000


def _gv_block(ns, nv48, wv1t, wv2t, wsvt, wsst, wgt, bg):
    fvs = []
    n2 = None
    for c in range(3):
        fvi = jnp.dot(nv48[:, c * 16:(c + 1) * 16], wv1t,
                      preferred_element_type=jnp.float32)
        fvs.append(fvi)
        n2 = fvi * fvi if n2 is None else n2 + fvi * fvi
    fvn = jnp.sqrt(n2)
    out_s = (jnp.dot(fvn, wsvt, preferred_element_type=jnp.float32)
             + jnp.dot(ns, wsst, preferred_element_type=jnp.float32))
    gate = jax.nn.sigmoid(jnp.dot(out_s, wgt, preferred_element_type=jnp.float32) + bg)
    out_v = jnp.concatenate(
        [gate * jnp.dot(fvs[c], wv2t, preferred_element_type=jnp.float32)
         for c in range(3)], axis=1)
    return out_s, out_v


def _stage_a_body(ns_ref, nv_ref, pos_ref,
                  nm_wv1t, nm_wv2t, nm_wsvt, nm_wsst, nm_wgt, nm_bg,
                  rc_wv1t, rc_wv2t, rc_wsvt, rc_wsst, rc_wgt, rc_bg,
                  wnsst, bnss, wnsvt, bnsv,
                  t_ref, pos8_ref, rs_ref, rv_ref):
    ns = ns_ref[...]
    nv = nv_ref[...]
    s_m, v_m = _gv_block(ns, nv, nm_wv1t[...], nm_wv2t[...], nm_wsvt[...],
                         nm_wsst[...], nm_wgt[...], nm_bg[...])
    p1 = jnp.dot(s_m, wnsst[...], preferred_element_type=jnp.float32) + bnss[...]
    p2 = jnp.dot(s_m, wnsvt[...], preferred_element_type=jnp.float32) + bnsv[...]
    rs, rv = _gv_block(ns, nv, rc_wv1t[...], rc_wv2t[...], rc_wsvt[...],
                       rc_wsst[...], rc_wgt[...], rc_bg[...])
    pos = pos_ref[...]
    b = ns.shape[0]
    zeros13 = jnp.zeros((b, 13), jnp.float32)
    t_ref[...] = jnp.concatenate([p1, p2, v_m, pos, zeros13], axis=1)
    pos8_ref[...] = jnp.concatenate([pos, jnp.zeros((b, 5), jnp.float32)], axis=1)
    rs_ref[...] = rs
    rv_ref[...] = rv


def _edge_body(g_ref, pc8_ref, gds_ref, feat_ref,
               off_ref, wsst_e, c0_ref, wget, bge, bvec, wevvt,
               wesst, bess, wesvt, besv, wnvvt,
               wv1mt, wsmvt, wsmst, wv2mt, wgmt, bgm,
               m0_ref, m1_ref, m2_ref, m3_ref):
    g = g_ref[...]
    p1 = g[:, 0:32]
    p2 = g[:, 32:64]
    posr = g[:, 112:115]
    posc = pc8_ref[:, 0:3]
    d = posr - posc
    r2 = jnp.sum(d * d, axis=1, keepdims=True)
    r = jnp.sqrt(r2)
    inv = 1.0 / (r + 1e-7)
    evn = d * inv
    s_e = r * inv
    gds = gds_ref[...]
    cexp = -0.5 / (_CUTOFF / (_NUM_GAUSS - 1)) ** 2
    dd = gds - off_ref[...]
    smear = jnp.exp(cexp * dd * dd)
    esf = jnp.concatenate([smear, feat_ref[...]], axis=1)
    edge_s = (jnp.dot(esf, wsst_e[...], preferred_element_type=jnp.float32)
              + s_e * c0_ref[...])
    gate_e = jax.nn.sigmoid(
        jnp.dot(edge_s, wget[...], preferred_element_type=jnp.float32) + bge[...])
    gb = gate_e * bvec[...]
    g2 = jnp.dot(gb, wevvt[...], preferred_element_type=jnp.float32)
    q0 = jnp.dot(edge_s, wesst[...], preferred_element_type=jnp.float32) + bess[...]
    q1 = jnp.dot(edge_s, wesvt[...], preferred_element_type=jnp.float32) + besv[...]
    coeff = 0.5 * (jnp.cos(gds * (math.pi / _CUTOFF)) + 1.0)
    coeff = jnp.where((gds <= _CUTOFF) & (gds >= 0.0), coeff, 0.0)
    msg_s = p1 * q0 * coeff
    p2g2 = p2 * g2
    fvs = []
    fvn2 = None
    for c in range(3):
        nv_c = g[:, 64 + 16 * c: 64 + 16 * (c + 1)]
        nv2c = jnp.dot(nv_c, wnvvt[...], preferred_element_type=jnp.float32)
        mvc = coeff * (nv2c * q1 + p2g2 * evn[:, c:c + 1])
        fvc = jnp.dot(mvc, wv1mt[...], preferred_element_type=jnp.float32)
        fvs.append(fvc)
        fvn2 = fvc * fvc if fvn2 is None else fvn2 + fvc * fvc
    fvn = jnp.sqrt(fvn2)
    out_s = (jnp.dot(fvn, wsmvt[...], preferred_element_type=jnp.float32)
             + jnp.dot(msg_s, wsmst[...], preferred_element_type=jnp.float32))
    gate_m = jax.nn.sigmoid(
        jnp.dot(out_s, wgmt[...], preferred_element_type=jnp.float32) + bgm[...])
    mv = [gate_m * jnp.dot(fvs[c], wv2mt[...], preferred_element_type=jnp.float32)
          for c in range(3)]
    m0_ref[...] = out_s[:, 0:32]
    m1_ref[...] = out_s[:, 32:64]
    m2_ref[...] = jnp.concatenate([mv[0], mv[1]], axis=1)
    m3_ref[...] = mv[2]


def _stage_d_body(a0_ref, a1_ref, a2_ref, a3_ref, rs_ref, rv_ref,
                  ao_wv1t, ao_wv2t, ao_wsvt, ao_wsst, ao_wgt, ao_bg,
                  os_ref, ov_ref):
    in_s = rs_ref[...] + jnp.concatenate([a0_ref[...], a1_ref[...]], axis=1)
    in_v = rv_ref[...] + jnp.concatenate([a2_ref[...], a3_ref[...]], axis=1)
    out_s, out_v = _gv_block(in_s, in_v, ao_wv1t[...], ao_wv2t[...], ao_wsvt[...],
                             ao_wsst[...], ao_wgt[...], ao_bg[...])
    os_ref[...] = out_s
    ov_ref[...] = out_v


def _full_spec(shape):
    return pl.BlockSpec(shape, lambda i: tuple(0 for _ in shape))


def _row_spec(block_rows, cols):
    return pl.BlockSpec((block_rows, cols), lambda i: (i, 0))


def kernel(node_sca, node_vec, node_pos, edge_feature, edge_index, gds_dist, params):
    n = node_sca.shape[0]
    e = gds_dist.shape[0]
    p = params
    f32 = jnp.float32

    nv48 = node_vec.transpose(0, 2, 1).reshape(n, 48)

    nm, rc, mo, ao = (p['node_mapper'], p['resi_connecter'], p['msg_out'],
                      p['aggr_out'])
    em = p['edge_mapper']

    w_exp = em['Wv1'] @ p['edge_expansion'][:, 0]
    b_vec = em['Wv2'] @ w_exp
    c0 = em['Ws'][:, :16] @ jnp.abs(w_exp)
    offs = jnp.asarray(np.linspace(0.0, _CUTOFF, _NUM_GAUSS), f32)

    def gv_args(q):
        return (q['Wv1'].T, q['Wv2'].T, q['Ws'][:, :16].T, q['Ws'][:, 16:].T,
                q['Wg'].T, q['bg'][None, :])

    grid_a = (n // _N_BLOCK,)
    a_weights = (*gv_args(nm), *gv_args(rc),
                 p['node_sca_sca']['W'].T, p['node_sca_sca']['b'][None, :],
                 p['node_sca_vec']['W'].T, p['node_sca_vec']['b'][None, :])
    a_wspecs = [_full_spec(w.shape) for w in a_weights]
    t_tab, pos8, resi_s, resi_v = pl.pallas_call(
        _stage_a_body,
        grid=grid_a,
        in_specs=[_row_spec(_N_BLOCK, 64), _row_spec(_N_BLOCK, 48),
                  _row_spec(_N_BLOCK, 3)] + a_wspecs,
        out_specs=[_row_spec(_N_BLOCK, 128), _row_spec(_N_BLOCK, 8),
                   _row_spec(_N_BLOCK, 64), _row_spec(_N_BLOCK, 48)],
        out_shape=[jax.ShapeDtypeStruct((n, 128), f32),
                   jax.ShapeDtypeStruct((n, 8), f32),
                   jax.ShapeDtypeStruct((n, 64), f32),
                   jax.ShapeDtypeStruct((n, 48), f32)],
    )(node_sca, nv48, node_pos, *a_weights)

    row = edge_index[0]
    col = edge_index[1]
    g_rows = jnp.take(t_tab, row, axis=0)
    pc8 = jnp.take(pos8, col, axis=0)

    e_weights = (offs[None, :], em['Ws'][:, 16:].T, c0[None, :], em['Wg'].T,
                 em['bg'][None, :], b_vec[None, :], p['edge_vec_vec'].T,
                 p['edge_sca_sca']['W'].T, p['edge_sca_sca']['b'][None, :],
                 p['edge_sca_vec']['W'].T, p['edge_sca_vec']['b'][None, :],
                 p['node_vec_vec'].T,
                 mo['Wv1'].T, mo['Ws'][:, :32].T, mo['Ws'][:, 32:].T,
                 mo['Wv2'].T, mo['Wg'].T, mo['bg'][None, :])
    e_wspecs = [_full_spec(w.shape) for w in e_weights]
    grid_e = (e // _E_BLOCK,)
    m0, m1, m2, m3 = pl.pallas_call(
        _edge_body,
        grid=grid_e,
        in_specs=[_row_spec(_E_BLOCK, 128), _row_spec(_E_BLOCK, 8),
                  _row_spec(_E_BLOCK, 1), _row_spec(_E_BLOCK, 2)] + e_wspecs,
        out_specs=[_row_spec(_E_BLOCK, 32), _row_spec(_E_BLOCK, 32),
                   _row_spec(_E_BLOCK, 32), _row_spec(_E_BLOCK, 16)],
        out_shape=[jax.ShapeDtypeStruct((e, 32), f32),
                   jax.ShapeDtypeStruct((e, 32), f32),
                   jax.ShapeDtypeStruct((e, 32), f32),
                   jax.ShapeDtypeStruct((e, 16), f32)],
    )(g_rows, pc8, gds_dist[:, None], edge_feature, *e_weights)

    a0 = jax.ops.segment_sum(m0, row, num_segments=n)
    a1 = jax.ops.segment_sum(m1, row, num_segments=n)
    a2 = jax.ops.segment_sum(m2, row, num_segments=n)
    a3 = jax.ops.segment_sum(m3, row, num_segments=n)

    d_weights = gv_args(ao)
    d_wspecs = [_full_spec(w.shape) for w in d_weights]
    out_s, out_v48 = pl.pallas_call(
        _stage_d_body,
        grid=grid_a,
        in_specs=[_row_spec(_N_BLOCK, 32), _row_spec(_N_BLOCK, 32),
                  _row_spec(_N_BLOCK, 32), _row_spec(_N_BLOCK, 16),
                  _row_spec(_N_BLOCK, 64), _row_spec(_N_BLOCK, 48)] + d_wspecs,
        out_specs=[_row_spec(_N_BLOCK, 64), _row_spec(_N_BLOCK, 48)],
        out_shape=[jax.ShapeDtypeStruct((n, 64), f32),
                   jax.ShapeDtypeStruct((n, 48), f32)],
    )(a0, a1, a2, a3, resi_s, resi_v, *d_weights)

    out_v = out_v48.reshape(n, 3, 16).transpose(0, 2, 1)
    return out_s, out_v
